# Initial kernel scaffold; baseline (speedup 1.0000x reference)
import jax
import jax.numpy as jnp
from jax.experimental import pallas as pl


def kernel(X, edge_index, edge_weight, H, Wxz, bxz, Whz, bhz, Wxr, bxr, Whr, bhr, Wxh, bxh, Whh, bhh):
    raise NotImplementedError("write your pallas kernel here")


# speedup vs baseline: 5.0936x; 5.0936x over previous
import functools

import jax
import jax.numpy as jnp
from jax import lax
from jax.experimental import pallas as pl
from jax.experimental.pallas import tpu as pltpu
from jax.experimental.pallas import tpu_sc as plsc

N = 10000
E = 320000
D = 128
E_PAD = 327680
EROWS = E_PAD // 128
NTILES = 16


def _zeros16f():
    return jnp.zeros((16,), jnp.float32)


def _bcast_lane(v, j):
    return jnp.take(v, jnp.full((16,), j, jnp.int32))


def _zero_rows(rows_t, nrows, ngrp):
    def body(r, _):
        for g in range(ngrp):
            rows_t[r, pl.ds(g * 16, 16)] = _zeros16f()
        return 0
    lax.fori_loop(0, nrows, body, 0, unroll=8)


def _zero_acc_slice(acc, rows2, s):
    base = pl.multiple_of(s * 624, 8)
    for off in (0, 128, 256, 384):
        pltpu.sync_copy(rows2.at[0], acc.at[pl.ds(base + off, 128)])
    pltpu.sync_copy(rows2.at[0, pl.ds(0, 112)], acc.at[pl.ds(base + 512, 112)])

    @pl.when(s == NTILES - 1)
    def _():
        pltpu.sync_copy(rows2.at[0, pl.ds(0, 16)], acc.at[pl.ds(9984, 16)])


def _writeout(acc, out_ref, c, s):
    base = pl.multiple_of(s * 624, 8)
    pltpu.sync_copy(acc.at[pl.ds(base, 624)], out_ref.at[c, pl.ds(base, 624)])

    @pl.when(s == NTILES - 1)
    def _():
        pltpu.sync_copy(acc.at[pl.ds(9984, 16)], out_ref.at[c, pl.ds(9984, 16)])


def _make_deg():
    mesh = plsc.VectorSubcoreMesh(core_axis_name="c", subcore_axis_name="s")

    @functools.partial(
        pl.kernel, mesh=mesh,
        out_type=jax.ShapeDtypeStruct((2, N), jnp.float32),
        scratch_types=dict(
            src2=pltpu.VMEM((80, 128), jnp.int32),
            w2=pltpu.VMEM((80, 128), jnp.float32),
            z=pltpu.VMEM((640,), jnp.float32),
            sdeg=pltpu.VMEM_SHARED((N,), jnp.float32),
        ),
    )
    def deg_kernel(srcr, wmr, deg_out, src2, w2, z, sdeg):
        c = lax.axis_index("c")
        s = lax.axis_index("s")
        rb = (c * NTILES + s) * 80
        pltpu.sync_copy(srcr.at[pl.ds(rb, 80)], src2)
        pltpu.sync_copy(wmr.at[pl.ds(rb, 80)], w2)

        def zb(r, _):
            z[pl.ds(r * 16, 16)] = _zeros16f()
            return 0
        lax.fori_loop(0, 40, zb, 0)
        base = pl.multiple_of(s * 624, 8)
        pltpu.sync_copy(z.at[pl.ds(0, 624)], sdeg.at[pl.ds(base, 624)])

        @pl.when(s == NTILES - 1)
        def _():
            pltpu.sync_copy(z.at[pl.ds(0, 16)], sdeg.at[pl.ds(9984, 16)])
        plsc.subcore_barrier()

        def blk(b, _):
            pltpu.sync_copy(w2.at[b], sdeg.at[src2.at[b]], add=True)
            return 0
        lax.fori_loop(0, 80, blk, 0)
        plsc.subcore_barrier()

        @pl.when(s == 0)
        def _():
            pltpu.sync_copy(sdeg, deg_out.at[c])

    return deg_kernel


def _make_hop(full_e):
    mesh = plsc.VectorSubcoreMesh(core_axis_name="c", subcore_axis_name="s")
    nbatch = 20 if full_e else 10

    @functools.partial(
        pl.kernel, mesh=mesh,
        out_type=jax.ShapeDtypeStruct((2, N, D), jnp.float32),
        scratch_types=dict(
            eb=pltpu.VMEM((2, 8, 2, 128), jnp.int32),
            wbuf=pltpu.VMEM((2, 8, 128), jnp.float32),
            rows2=pltpu.VMEM((2, 128, D), jnp.float32),
            acc=pltpu.VMEM_SHARED((N, D), jnp.float32),
            semb=pltpu.SemaphoreType.DMA,
            semg=pltpu.SemaphoreType.DMA,
        ),
    )
    def hop(source, e3, wmr, out, eb, wbuf, rows2, acc, semb, semg):
        c = lax.axis_index("c")
        s = lax.axis_index("s")
        rb = (c * EROWS + s * 160) if full_e else (c * NTILES + s) * 80
        wb = s * 160 if full_e else (c * NTILES + s) * 80
        def zr(r, _):
            for g in range(D // 16):
                rows2[0, r, pl.ds(g * 16, 16)] = _zeros16f()
            return 0
        lax.fori_loop(0, 128, zr, 0, unroll=8)
        _zero_acc_slice(acc, rows2, s)
        plsc.subcore_barrier()

        pltpu.async_copy(e3.at[pl.ds(rb, 8)], eb.at[0], semb)
        pltpu.async_copy(wmr.at[pl.ds(wb, 8)], wbuf.at[0], semb)

        def batch(t, _):
            slot = lax.rem(t, 2)
            pltpu.make_async_copy(e3.at[pl.ds(rb, 8)], eb.at[slot], semb).wait()
            pltpu.make_async_copy(wmr.at[pl.ds(wb, 8)], wbuf.at[slot],
                                  semb).wait()

            @pl.when(t < nbatch - 1)
            def _():
                pltpu.async_copy(e3.at[pl.ds(rb + (t + 1) * 8, 8)],
                                 eb.at[1 - slot], semb)
                pltpu.async_copy(wmr.at[pl.ds(wb + (t + 1) * 8, 8)],
                                 wbuf.at[1 - slot], semb)

            pltpu.async_copy(source.at[eb.at[slot, 0, 0]], rows2.at[0], semg)

            for b in range(8):
                rs = b & 1
                pltpu.make_async_copy(source.at[eb.at[slot, b, 0]],
                                      rows2.at[rs], semg).wait()
                if b < 7:
                    pltpu.async_copy(source.at[eb.at[slot, b + 1, 0]],
                                     rows2.at[1 - rs], semg)

                def grp(g, _):
                    wv = wbuf[slot, b, pl.ds(g * 16, 16)]
                    for j in range(16):
                        e = g * 16 + j
                        wb = _bcast_lane(wv, j)
                        for f in range(D // 16):
                            sl = pl.ds(f * 16, 16)
                            rows2[rs, e, sl] = rows2[rs, e, sl] * wb
                    return 0
                lax.fori_loop(0, 8, grp, 0)
                pltpu.sync_copy(rows2.at[rs], acc.at[eb.at[slot, b, 1]],
                                add=True)
            return 0
        lax.fori_loop(0, nbatch, batch, 0)
        plsc.subcore_barrier()
        _writeout(acc, out, c, s)

    return hop


_deg = _make_deg()
_hop_full = _make_hop(True)
_hop_half = _make_hop(False)


_BLK = 1000


def _tc_prep_body(x, h, dg, uv_o, scl_o):
    deg = dg[0] + dg[1]
    pos = deg > 0.0
    safe = jnp.where(pos, deg, 1.0)
    dinv = jnp.where(pos, lax.rsqrt(safe), 0.0)
    dinv2 = jnp.where(pos, 1.0 / safe, 0.0)
    dinvb = jnp.broadcast_to(dinv, (_BLK, D))
    uv_o[0] = dinvb * x[...]
    uv_o[1] = dinvb * h[...]
    scl_o[0] = dinvb
    scl_o[1] = jnp.broadcast_to(dinv2, (_BLK, D))


def _tc_prep(x, h, degp):
    row = pl.BlockSpec((_BLK, D), lambda i: (i, 0))
    two = pl.BlockSpec((2, _BLK, D), lambda i: (0, i, 0))
    return pl.pallas_call(
        _tc_prep_body,
        grid=(N // _BLK,),
        in_specs=[row, row, pl.BlockSpec((2, _BLK, 1), lambda i: (0, i, 0))],
        out_specs=[two, two],
        out_shape=[jax.ShapeDtypeStruct((2, N, D), jnp.float32)] * 2,
    )(x, h, degp)


def _tc_mid1_body(scl, t1, q_o):
    q_o[0] = scl[1] * t1[0]
    q_o[1] = scl[1] * t1[1]


def _tc_mid1(scl, t1):
    two = pl.BlockSpec((2, _BLK, D), lambda i: (0, i, 0))
    return pl.pallas_call(
        _tc_mid1_body,
        grid=(N // _BLK,),
        in_specs=[two, two],
        out_specs=two,
        out_shape=jax.ShapeDtypeStruct((2, N, D), jnp.float32),
    )(scl, t1)


def _tc_gates_body(x, h, t1, t2, scl, g, b3, z_o, hr_o, uhr_o, xh_o):
    dinvb = scl[0]
    ins = (x[...], dinvb * t1[0], dinvb * t2[0],
           h[...], dinvb * t1[1], dinvb * t2[1])
    acc = jnp.zeros((_BLK, 3 * D), jnp.float32)
    for k in range(6):
        acc = acc + lax.dot_general(
            ins[k], g[k], (((1,), (0,)), ((), ())),
            preferred_element_type=jnp.float32)
    acc = acc + b3[...]
    z = jax.nn.sigmoid(acc[:, 0:D])
    r = jax.nn.sigmoid(acc[:, D:2 * D])
    hr = h[...] * r
    z_o[...] = z
    hr_o[...] = hr
    uhr_o[...] = dinvb * hr
    xh_o[...] = acc[:, 2 * D:3 * D]


def _tc_gates(x, h, t1, t2, scl, g, b3):
    row = pl.BlockSpec((_BLK, D), lambda i: (i, 0))
    two = pl.BlockSpec((2, _BLK, D), lambda i: (0, i, 0))
    return pl.pallas_call(
        _tc_gates_body,
        grid=(N // _BLK,),
        in_specs=[row, row, two, two, two,
                  pl.BlockSpec((6, D, 3 * D), lambda i: (0, 0, 0)),
                  pl.BlockSpec((1, 3 * D), lambda i: (0, 0))],
        out_specs=[row, row, row, row],
        out_shape=[jax.ShapeDtypeStruct((N, D), jnp.float32)] * 4,
    )(x, h, t1, t2, scl, g, b3)


def _tc_mid2_body(scl, t1r, t1s_o, qr_o):
    t1s = t1r[0] + t1r[1]
    t1s_o[...] = t1s
    qr_o[...] = scl[1] * t1s


def _tc_mid2(scl, t1r):
    row = pl.BlockSpec((_BLK, D), lambda i: (i, 0))
    two = pl.BlockSpec((2, _BLK, D), lambda i: (0, i, 0))
    return pl.pallas_call(
        _tc_mid2_body,
        grid=(N // _BLK,),
        in_specs=[two, two],
        out_specs=[row, row],
        out_shape=[jax.ShapeDtypeStruct((N, D), jnp.float32)] * 2,
    )(scl, t1r)


def _tc_final_body(z, h, xh, hr, t1s, t2r, scl, gh, bh, o):
    dinvb = scl[0]
    ins = (hr[...], dinvb * t1s[...], dinvb * (t2r[0] + t2r[1]))
    acc = xh[...] + bh[...]
    for k in range(3):
        acc = acc + lax.dot_general(
            ins[k], gh[k], (((1,), (0,)), ((), ())),
            preferred_element_type=jnp.float32)
    ht = jnp.tanh(acc)
    zz = z[...]
    o[...] = zz * h[...] + (1.0 - zz) * ht


def _tc_final(z, h, xh, hr, t1s, t2r, scl):
    def call(gh, bh):
        row = pl.BlockSpec((_BLK, D), lambda i: (i, 0))
        two = pl.BlockSpec((2, _BLK, D), lambda i: (0, i, 0))
        return pl.pallas_call(
            _tc_final_body,
            grid=(N // _BLK,),
            in_specs=[row, row, row, row, row, two, two,
                      pl.BlockSpec((3, D, D), lambda i: (0, 0, 0)),
                      pl.BlockSpec((1, D), lambda i: (0, 0))],
            out_specs=row,
            out_shape=jax.ShapeDtypeStruct((N, D), jnp.float32),
        )(z, h, xh, hr, t1s, t2r, scl, gh, bh)
    return call


def kernel(X, edge_index, edge_weight, H, Wxz, bxz, Whz, bhz, Wxr, bxr,
           Whr, bhr, Wxh, bxh, Whh, bhh):
    src = edge_index[0]
    dst = edge_index[1]
    pad = E_PAD - E
    srcr = jnp.concatenate([src, jnp.zeros((pad,), src.dtype)]).reshape(EROWS, 128)
    dstr = jnp.concatenate([dst, jnp.zeros((pad,), dst.dtype)]).reshape(EROWS, 128)
    wm = jnp.where(src != dst, edge_weight, 0.0)
    wmr = jnp.concatenate(
        [wm, jnp.zeros((pad,), edge_weight.dtype)]).reshape(EROWS, 128)
    e3 = jnp.concatenate([
        jnp.stack([srcr, dstr], axis=1),
        jnp.stack([srcr + N, dstr], axis=1),
    ])

    def fold(w):
        return w[0] - w[2], -w[1], 2.0 * w[2]

    axz, bxz_, cxz = fold(Wxz)
    ahz, bhz_, chz = fold(Whz)
    axr, bxr_, cxr = fold(Wxr)
    ahr, bhr_, chr_ = fold(Whr)
    axh, bxh_, cxh = fold(Wxh)
    ahh, bhh_, chh = fold(Whh)
    g = jnp.stack([
        jnp.concatenate([axz, axr, axh], axis=1),
        jnp.concatenate([bxz_, bxr_, bxh_], axis=1),
        jnp.concatenate([cxz, cxr, cxh], axis=1),
        jnp.concatenate([ahz, ahr, jnp.zeros((D, D), jnp.float32)], axis=1),
        jnp.concatenate([bhz_, bhr_, jnp.zeros((D, D), jnp.float32)], axis=1),
        jnp.concatenate([chz, chr_, jnp.zeros((D, D), jnp.float32)], axis=1),
    ])
    b3 = jnp.concatenate([bxz + bhz, bxr + bhr, bxh]).reshape(1, 3 * D)
    gh = jnp.stack([ahh, bhh_, chh])
    bhv = bhh.reshape(1, D)

    degp = _deg(srcr, wmr)
    uv, scl = _tc_prep(X, H, degp.reshape(2, N, 1))
    t1 = _hop_full(uv.reshape(2 * N, D), e3, wmr)
    q = _tc_mid1(scl, t1)
    t2 = _hop_full(q.reshape(2 * N, D), e3, wmr)
    z, hr, uhr, xh_part = _tc_gates(X, H, t1, t2, scl, g, b3)
    t1r_parts = _hop_half(uhr, e3, wmr)
    t1s, qr = _tc_mid2(scl, t1r_parts)
    t2r_parts = _hop_half(qr, e3, wmr)
    return _tc_final(z, H, xh_part, hr, t1s, t2r_parts, scl)(gh, bhv)


# speedup vs baseline: 5.8634x; 1.1511x over previous
import functools

import jax
import jax.numpy as jnp
from jax import lax
from jax.experimental import pallas as pl
from jax.experimental.pallas import tpu as pltpu
from jax.experimental.pallas import tpu_sc as plsc

N = 10000
E = 320000
D = 128
E_PAD = 327680
EROWS = E_PAD // 128
NTILES = 16


def _zeros16f():
    return jnp.zeros((16,), jnp.float32)


def _bcast_lane(v, j):
    return jnp.take(v, jnp.full((16,), j, jnp.int32))


def _zero_rows(rows_t, nrows, ngrp):
    def body(r, _):
        for g in range(ngrp):
            rows_t[r, pl.ds(g * 16, 16)] = _zeros16f()
        return 0
    lax.fori_loop(0, nrows, body, 0, unroll=8)


def _zero_acc_slice(acc, rows2, s):
    base = pl.multiple_of(s * 624, 8)
    for off in (0, 128, 256, 384):
        pltpu.sync_copy(rows2.at[0], acc.at[pl.ds(base + off, 128)])
    pltpu.sync_copy(rows2.at[0, pl.ds(0, 112)], acc.at[pl.ds(base + 512, 112)])

    @pl.when(s == NTILES - 1)
    def _():
        pltpu.sync_copy(rows2.at[0, pl.ds(0, 16)], acc.at[pl.ds(9984, 16)])


def _writeout(acc, out_ref, c, s):
    base = pl.multiple_of(s * 624, 8)
    pltpu.sync_copy(acc.at[pl.ds(base, 624)], out_ref.at[c, pl.ds(base, 624)])

    @pl.when(s == NTILES - 1)
    def _():
        pltpu.sync_copy(acc.at[pl.ds(9984, 16)], out_ref.at[c, pl.ds(9984, 16)])


def _make_deg():
    mesh = plsc.VectorSubcoreMesh(core_axis_name="c", subcore_axis_name="s")

    @functools.partial(
        pl.kernel, mesh=mesh,
        out_type=jax.ShapeDtypeStruct((2, N), jnp.float32),
        scratch_types=dict(
            src2=pltpu.VMEM((80, 128), jnp.int32),
            w2=pltpu.VMEM((80, 128), jnp.float32),
            z=pltpu.VMEM((640,), jnp.float32),
            sdeg=pltpu.VMEM_SHARED((N,), jnp.float32),
        ),
    )
    def deg_kernel(srcr, wmr, deg_out, src2, w2, z, sdeg):
        c = lax.axis_index("c")
        s = lax.axis_index("s")
        rb = (c * NTILES + s) * 80
        pltpu.sync_copy(srcr.at[pl.ds(rb, 80)], src2)
        pltpu.sync_copy(wmr.at[pl.ds(rb, 80)], w2)

        def zb(r, _):
            z[pl.ds(r * 16, 16)] = _zeros16f()
            return 0
        lax.fori_loop(0, 40, zb, 0)
        base = pl.multiple_of(s * 624, 8)
        pltpu.sync_copy(z.at[pl.ds(0, 624)], sdeg.at[pl.ds(base, 624)])

        @pl.when(s == NTILES - 1)
        def _():
            pltpu.sync_copy(z.at[pl.ds(0, 16)], sdeg.at[pl.ds(9984, 16)])
        plsc.subcore_barrier()

        def blk(b, _):
            pltpu.sync_copy(w2.at[b], sdeg.at[src2.at[b]], add=True)
            return 0
        lax.fori_loop(0, 80, blk, 0)
        plsc.subcore_barrier()

        @pl.when(s == 0)
        def _():
            pltpu.sync_copy(sdeg, deg_out.at[c])

    return deg_kernel


def _make_hop(width, scaled_out):
    mesh = plsc.VectorSubcoreMesh(core_axis_name="c", subcore_axis_name="s")
    full_e = width == 128
    nbatch = 10 if full_e else 5
    bpb = 16
    w = 128
    nf = w // 16
    outs = [jax.ShapeDtypeStruct((2, N, w), jnp.float32)]
    if scaled_out:
        outs.append(jax.ShapeDtypeStruct((2, N, w), jnp.float32))

    scratch = dict(
        eb=pltpu.VMEM((2, bpb, 2, 128), jnp.int32),
        wbuf=pltpu.VMEM((2, bpb, 128), jnp.float32),
        rows2=pltpu.VMEM((2, 128, w), jnp.float32),
        acc=pltpu.VMEM_SHARED((N, w), jnp.float32),
        semb=pltpu.SemaphoreType.DMA,
        semg=pltpu.SemaphoreType.DMA,
        sems=pltpu.SemaphoreType.DMA,
    )

    def body(source, e3, wmr, d2fs, out, qout, eb, wbuf, rows2, acc,
             semb, semg, sems):
        c = lax.axis_index("c")
        s = lax.axis_index("s")
        if full_e:
            rb = c * EROWS + s * 160
            wb0 = s * 160
        else:
            rb = (c * NTILES + s) * 80
            wb0 = rb

        def zr(r, _):
            for g in range(nf):
                rows2[0, r, pl.ds(g * 16, 16)] = _zeros16f()
            return 0
        lax.fori_loop(0, 128, zr, 0, unroll=8)
        _zero_acc_slice(acc, rows2, s)
        plsc.subcore_barrier()

        pltpu.async_copy(e3.at[pl.ds(rb, bpb)], eb.at[0], semb)
        pltpu.async_copy(wmr.at[pl.ds(wb0, bpb)], wbuf.at[0], semb)

        def batch(t, _):
            slot = lax.rem(t, 2)
            pltpu.make_async_copy(e3.at[pl.ds(rb, bpb)], eb.at[slot],
                                  semb).wait()
            pltpu.make_async_copy(wmr.at[pl.ds(wb0, bpb)], wbuf.at[slot],
                                  semb).wait()

            @pl.when(t < nbatch - 1)
            def _():
                pltpu.async_copy(e3.at[pl.ds(rb + (t + 1) * bpb, bpb)],
                                 eb.at[1 - slot], semb)
                pltpu.async_copy(wmr.at[pl.ds(wb0 + (t + 1) * bpb, bpb)],
                                 wbuf.at[1 - slot], semb)

            pltpu.async_copy(source.at[eb.at[slot, 0, 0]], rows2.at[0], semg)

            for b in range(bpb):
                rs = b & 1
                pltpu.make_async_copy(source.at[eb.at[slot, b, 0]],
                                      rows2.at[rs], semg).wait()
                if b > 0:
                    pltpu.make_async_copy(rows2.at[1 - rs],
                                          acc.at[eb.at[slot, b - 1, 1]],
                                          sems).wait()
                else:
                    @pl.when(t > 0)
                    def _():
                        pltpu.make_async_copy(rows2.at[1],
                                              acc.at[eb.at[slot, 0, 1]],
                                              sems).wait()
                if b < bpb - 1:
                    pltpu.async_copy(source.at[eb.at[slot, b + 1, 0]],
                                     rows2.at[1 - rs], semg)

                def grp(g, _):
                    wv = wbuf[slot, b, pl.ds(g * 16, 16)]
                    for j in range(16):
                        e = g * 16 + j
                        wbc = _bcast_lane(wv, j)
                        for f in range(nf):
                            sl = pl.ds(f * 16, 16)
                            rows2[rs, e, sl] = rows2[rs, e, sl] * wbc
                    return 0
                lax.fori_loop(0, 8, grp, 0)
                pltpu.async_copy(rows2.at[rs], acc.at[eb.at[slot, b, 1]],
                                 sems, add=True)
            return 0
        lax.fori_loop(0, nbatch, batch, 0)
        pltpu.make_async_copy(rows2.at[1], acc.at[eb.at[1, 0, 1]], sems).wait()
        plsc.subcore_barrier()

        base = pl.multiple_of(s * 624, 8)
        chunks = [(0, 128), (128, 128), (256, 128), (384, 128), (512, 112)]
        for off, ln in chunks:
            pltpu.sync_copy(acc.at[pl.ds(base + off, ln)],
                            out.at[c, pl.ds(base + off, ln)])

        @pl.when(s == NTILES - 1)
        def _():
            pltpu.sync_copy(acc.at[pl.ds(9984, 16)],
                            out.at[c, pl.ds(9984, 16)])

        if scaled_out:
            def scale_chunk(off, ln):
                pltpu.sync_copy(acc.at[pl.ds(base + off, ln)],
                                rows2.at[0, pl.ds(0, ln)])
                pltpu.sync_copy(d2fs.at[pl.ds(base + off, ln)],
                                rows2.at[1, pl.ds(0, ln)])

                def rw(r, _):
                    for f in range(nf):
                        sl = pl.ds(f * 16, 16)
                        rows2[0, r, sl] = rows2[0, r, sl] * rows2[1, r, sl]
                    return 0
                lax.fori_loop(0, ln, rw, 0, unroll=8)
                pltpu.sync_copy(rows2.at[0, pl.ds(0, ln)],
                                qout.at[c, pl.ds(base + off, ln)])
            for off, ln in chunks:
                scale_chunk(off, ln)

            @pl.when(s == NTILES - 1)
            def _():
                pltpu.sync_copy(acc.at[pl.ds(9984, 16)],
                                rows2.at[0, pl.ds(0, 16)])
                pltpu.sync_copy(d2fs.at[pl.ds(9984, 16)],
                                rows2.at[1, pl.ds(0, 16)])

                def rwt(r, _):
                    for f in range(nf):
                        sl = pl.ds(f * 16, 16)
                        rows2[0, r, sl] = rows2[0, r, sl] * rows2[1, r, sl]
                    return 0
                lax.fori_loop(0, 16, rwt, 0, unroll=8)
                pltpu.sync_copy(rows2.at[0, pl.ds(0, 16)],
                                qout.at[c, pl.ds(9984, 16)])

    if scaled_out:
        fn = body
    else:
        def fn(source, e3, wmr, d2fs, out, eb, wbuf, rows2, acc,
               semb, semg, sems):
            body(source, e3, wmr, d2fs, out, None, eb, wbuf, rows2, acc,
                 semb, semg, sems)

    return functools.partial(
        pl.kernel, mesh=mesh,
        out_type=tuple(outs) if scaled_out else outs[0],
        scratch_types=scratch,
    )(fn)


_deg = _make_deg()
_hop1 = _make_hop(128, True)
_hop2 = _make_hop(128, False)
_hop3 = _make_hop(64, False)
_hop4 = _make_hop(64, False)


_BLK = 1000


def _tc_prep_body(x, h, dg, uv_o, dinvb_o, d2w_o):
    deg = dg[0] + dg[1]
    pos = deg > 0.0
    safe = jnp.where(pos, deg, 1.0)
    dinv = jnp.where(pos, lax.rsqrt(safe), 0.0)
    dinv2 = jnp.where(pos, 1.0 / safe, 0.0)
    dinvb = jnp.broadcast_to(dinv, (_BLK, D))
    uv_o[0] = dinvb * x[...]
    uv_o[1] = dinvb * h[...]
    dinvb_o[...] = dinvb
    d2w_o[...] = jnp.broadcast_to(dinv2, (_BLK, D))


def _tc_prep(x, h, degp):
    row = pl.BlockSpec((_BLK, D), lambda i: (i, 0))
    two = pl.BlockSpec((2, _BLK, D), lambda i: (0, i, 0))
    return pl.pallas_call(
        _tc_prep_body,
        grid=(N // _BLK,),
        in_specs=[row, row, pl.BlockSpec((2, _BLK, 1), lambda i: (0, i, 0))],
        out_specs=[two, row, row],
        out_shape=[jax.ShapeDtypeStruct((2, N, D), jnp.float32),
                   jax.ShapeDtypeStruct((N, D), jnp.float32),
                   jax.ShapeDtypeStruct((N, D), jnp.float32)],
    )(x, h, degp)


def _tc_gates_body(x, h, t1, t2, dinvb_r, g, b3, z_o, hr_o, uhr_o, xh_o):
    dinvb = dinvb_r[...]
    ins = (x[...], dinvb * t1[0], dinvb * t2[0],
           h[...], dinvb * t1[1], dinvb * t2[1])
    acc = jnp.zeros((_BLK, 3 * D), jnp.float32)
    for k in range(6):
        acc = acc + lax.dot_general(
            ins[k], g[k], (((1,), (0,)), ((), ())),
            preferred_element_type=jnp.float32)
    acc = acc + b3[...]
    z = jax.nn.sigmoid(acc[:, 0:D])
    r = jax.nn.sigmoid(acc[:, D:2 * D])
    hr = h[...] * r
    z_o[...] = z
    hr_o[...] = hr
    uhr_o[...] = dinvb * hr
    xh_o[...] = acc[:, 2 * D:3 * D]


def _tc_gates(x, h, t1, t2, dinvb, g, b3):
    row = pl.BlockSpec((_BLK, D), lambda i: (i, 0))
    two = pl.BlockSpec((2, _BLK, D), lambda i: (0, i, 0))
    return pl.pallas_call(
        _tc_gates_body,
        grid=(N // _BLK,),
        in_specs=[row, row, two, two, row,
                  pl.BlockSpec((6, D, 3 * D), lambda i: (0, 0, 0)),
                  pl.BlockSpec((1, 3 * D), lambda i: (0, 0))],
        out_specs=[row, row, row, row],
        out_shape=[jax.ShapeDtypeStruct((N, D), jnp.float32)] * 4,
    )(x, h, t1, t2, dinvb, g, b3)


def _tc_mid2_body(d2w, t1r, t1s_o, qr_o):
    t1s = t1r[0] + t1r[1]
    t1s_o[...] = t1s
    qr_o[...] = d2w[...] * t1s


def _tc_mid2(d2w, t1r):
    row = pl.BlockSpec((_BLK, D), lambda i: (i, 0))
    two = pl.BlockSpec((2, _BLK, D), lambda i: (0, i, 0))
    return pl.pallas_call(
        _tc_mid2_body,
        grid=(N // _BLK,),
        in_specs=[row, two],
        out_specs=[row, row],
        out_shape=[jax.ShapeDtypeStruct((N, D), jnp.float32)] * 2,
    )(d2w, t1r)


def _tc_final_body(z, h, xh, hr, t1s, t2r, dinvb_r, gh, bh, o):
    dinvb = dinvb_r[...]
    ins = (hr[...], dinvb * t1s[...], dinvb * (t2r[0] + t2r[1]))
    acc = xh[...] + bh[...]
    for k in range(3):
        acc = acc + lax.dot_general(
            ins[k], gh[k], (((1,), (0,)), ((), ())),
            preferred_element_type=jnp.float32)
    ht = jnp.tanh(acc)
    zz = z[...]
    o[...] = zz * h[...] + (1.0 - zz) * ht


def _tc_final(z, h, xh, hr, t1s, t2r, dinvb, gh, bh):
    row = pl.BlockSpec((_BLK, D), lambda i: (i, 0))
    two = pl.BlockSpec((2, _BLK, D), lambda i: (0, i, 0))
    return pl.pallas_call(
        _tc_final_body,
        grid=(N // _BLK,),
        in_specs=[row, row, row, row, row, two, row,
                  pl.BlockSpec((3, D, D), lambda i: (0, 0, 0)),
                  pl.BlockSpec((1, D), lambda i: (0, 0))],
        out_specs=row,
        out_shape=jax.ShapeDtypeStruct((N, D), jnp.float32),
    )(z, h, xh, hr, t1s, t2r, dinvb, gh, bh)


def kernel(X, edge_index, edge_weight, H, Wxz, bxz, Whz, bhz, Wxr, bxr,
           Whr, bhr, Wxh, bxh, Whh, bhh):
    src = edge_index[0]
    dst = edge_index[1]
    pad = E_PAD - E
    srcr = jnp.concatenate([src, jnp.zeros((pad,), src.dtype)]).reshape(EROWS, 128)
    dstr = jnp.concatenate([dst, jnp.zeros((pad,), dst.dtype)]).reshape(EROWS, 128)
    wm = jnp.where(src != dst, edge_weight, 0.0)
    wmr = jnp.concatenate(
        [wm, jnp.zeros((pad,), edge_weight.dtype)]).reshape(EROWS, 128)
    e3 = jnp.concatenate([
        jnp.stack([srcr, dstr], axis=1),
        jnp.stack([srcr + N, dstr], axis=1),
    ])

    def fold(w):
        return w[0] - w[2], -w[1], 2.0 * w[2]

    axz, bxz_, cxz = fold(Wxz)
    ahz, bhz_, chz = fold(Whz)
    axr, bxr_, cxr = fold(Wxr)
    ahr, bhr_, chr_ = fold(Whr)
    axh, bxh_, cxh = fold(Wxh)
    ahh, bhh_, chh = fold(Whh)
    g = jnp.stack([
        jnp.concatenate([axz, axr, axh], axis=1),
        jnp.concatenate([bxz_, bxr_, bxh_], axis=1),
        jnp.concatenate([cxz, cxr, cxh], axis=1),
        jnp.concatenate([ahz, ahr, jnp.zeros((D, D), jnp.float32)], axis=1),
        jnp.concatenate([bhz_, bhr_, jnp.zeros((D, D), jnp.float32)], axis=1),
        jnp.concatenate([chz, chr_, jnp.zeros((D, D), jnp.float32)], axis=1),
    ])
    b3 = jnp.concatenate([bxz + bhz, bxr + bhr, bxh]).reshape(1, 3 * D)
    gh = jnp.stack([ahh, bhh_, chh])
    bhv = bhh.reshape(1, D)

    degp = _deg(srcr, wmr)
    uv, dinvb, d2w = _tc_prep(X, H, degp.reshape(2, N, 1))
    t1, q = _hop1(uv.reshape(2 * N, D), e3, wmr, d2w)
    t2 = _hop2(q.reshape(2 * N, D), e3, wmr, d2w)
    z, hr, uhr, xh_part = _tc_gates(X, H, t1, t2, dinvb, g, b3)
    t1r_parts = _hop3(uhr, e3, wmr, d2w)
    t1s, qr = _tc_mid2(d2w, t1r_parts)
    t2r_parts = _hop4(qr, e3, wmr, d2w)
    return _tc_final(z, H, xh_part, hr, t1s, t2r_parts, dinvb, gh, bhv)


# speedup vs baseline: 13.2120x; 2.2533x over previous
import functools

import jax
import jax.numpy as jnp
from jax import lax
from jax.experimental import pallas as pl
from jax.experimental.pallas import tpu as pltpu
from jax.experimental.pallas import tpu_sc as plsc

N = 10000
E = 320000
D = 128
E_PAD = 327680
EROWS = E_PAD // 128
NTILES = 16


def _zeros16f():
    return jnp.zeros((16,), jnp.float32)


def _bcast_lane(v, j):
    return jnp.take(v, jnp.full((16,), j, jnp.int32))


def _zero_rows(rows_t, nrows, ngrp):
    def body(r, _):
        for g in range(ngrp):
            rows_t[r, pl.ds(g * 16, 16)] = _zeros16f()
        return 0
    lax.fori_loop(0, nrows, body, 0, unroll=8)


def _zero_acc_slice(acc, rows2, s):
    base = pl.multiple_of(s * 624, 8)
    for off in (0, 128, 256, 384):
        pltpu.sync_copy(rows2.at[0], acc.at[pl.ds(base + off, 128)])
    pltpu.sync_copy(rows2.at[0, pl.ds(0, 112)], acc.at[pl.ds(base + 512, 112)])

    @pl.when(s == NTILES - 1)
    def _():
        pltpu.sync_copy(rows2.at[0, pl.ds(0, 16)], acc.at[pl.ds(9984, 16)])


def _writeout(acc, out_ref, c, s):
    base = pl.multiple_of(s * 624, 8)
    pltpu.sync_copy(acc.at[pl.ds(base, 624)], out_ref.at[c, pl.ds(base, 624)])

    @pl.when(s == NTILES - 1)
    def _():
        pltpu.sync_copy(acc.at[pl.ds(9984, 16)], out_ref.at[c, pl.ds(9984, 16)])


def _make_deg():
    mesh = plsc.VectorSubcoreMesh(core_axis_name="c", subcore_axis_name="s")

    @functools.partial(
        pl.kernel, mesh=mesh,
        out_type=jax.ShapeDtypeStruct((2, N), jnp.float32),
        scratch_types=dict(
            src2=pltpu.VMEM((80, 128), jnp.int32),
            w2=pltpu.VMEM((80, 128), jnp.float32),
            z=pltpu.VMEM((640,), jnp.float32),
            sdeg=pltpu.VMEM_SHARED((N,), jnp.float32),
        ),
    )
    def deg_kernel(srcr, wmr, deg_out, src2, w2, z, sdeg):
        c = lax.axis_index("c")
        s = lax.axis_index("s")
        rb = (c * NTILES + s) * 80
        pltpu.sync_copy(srcr.at[pl.ds(rb, 80)], src2)
        pltpu.sync_copy(wmr.at[pl.ds(rb, 80)], w2)

        def zb(r, _):
            z[pl.ds(r * 16, 16)] = _zeros16f()
            return 0
        lax.fori_loop(0, 40, zb, 0)
        base = pl.multiple_of(s * 624, 8)
        pltpu.sync_copy(z.at[pl.ds(0, 624)], sdeg.at[pl.ds(base, 624)])

        @pl.when(s == NTILES - 1)
        def _():
            pltpu.sync_copy(z.at[pl.ds(0, 16)], sdeg.at[pl.ds(9984, 16)])
        plsc.subcore_barrier()

        def blk(b, _):
            pltpu.sync_copy(w2.at[b], sdeg.at[src2.at[b]], add=True)
            return 0
        lax.fori_loop(0, 80, blk, 0)
        plsc.subcore_barrier()

        @pl.when(s == 0)
        def _():
            pltpu.sync_copy(sdeg, deg_out.at[c])

    return deg_kernel


def _make_hop(width, scaled_out):
    mesh = plsc.VectorSubcoreMesh(core_axis_name="c", subcore_axis_name="s")
    full_e = width == 128
    nbatch = 10 if full_e else 5
    bpb = 16
    w = 128
    nf = w // 16
    outs = [jax.ShapeDtypeStruct((2, N, w), jnp.float32)]
    if scaled_out:
        outs.append(jax.ShapeDtypeStruct((2, N, w), jnp.float32))

    scratch = dict(
        eb=pltpu.VMEM((2, bpb, 2, 128), jnp.int32),
        wbuf=pltpu.VMEM((2, bpb, 128), jnp.float32),
        rows2=pltpu.VMEM((2, 128, w), jnp.float32),
        acc=pltpu.VMEM_SHARED((N, w), jnp.float32),
        semb=pltpu.SemaphoreType.DMA,
        semg=pltpu.SemaphoreType.DMA,
        sems=pltpu.SemaphoreType.DMA,
    )

    def body(source, e3, wmr, d2fs, out, qout, eb, wbuf, rows2, acc,
             semb, semg, sems):
        c = lax.axis_index("c")
        s = lax.axis_index("s")
        if full_e:
            rb = c * EROWS + s * 160
            wb0 = s * 160
        else:
            rb = (c * NTILES + s) * 80
            wb0 = rb

        def zr(r, _):
            for g in range(nf):
                rows2[0, r, pl.ds(g * 16, 16)] = _zeros16f()
            return 0
        lax.fori_loop(0, 128, zr, 0, unroll=8)
        _zero_acc_slice(acc, rows2, s)
        plsc.subcore_barrier()

        pltpu.async_copy(e3.at[pl.ds(rb, bpb)], eb.at[0], semb)
        pltpu.async_copy(wmr.at[pl.ds(wb0, bpb)], wbuf.at[0], semb)

        def batch(t, _):
            slot = lax.rem(t, 2)
            pltpu.make_async_copy(e3.at[pl.ds(rb, bpb)], eb.at[slot],
                                  semb).wait()
            pltpu.make_async_copy(wmr.at[pl.ds(wb0, bpb)], wbuf.at[slot],
                                  semb).wait()

            @pl.when(t < nbatch - 1)
            def _():
                pltpu.async_copy(e3.at[pl.ds(rb + (t + 1) * bpb, bpb)],
                                 eb.at[1 - slot], semb)
                pltpu.async_copy(wmr.at[pl.ds(wb0 + (t + 1) * bpb, bpb)],
                                 wbuf.at[1 - slot], semb)

            pltpu.async_copy(source.at[eb.at[slot, 0, 0]], rows2.at[0], semg)

            for b in range(bpb):
                rs = b & 1
                pltpu.make_async_copy(source.at[eb.at[slot, b, 0]],
                                      rows2.at[rs], semg).wait()
                if b > 0:
                    pltpu.make_async_copy(rows2.at[1 - rs],
                                          acc.at[eb.at[slot, b - 1, 1]],
                                          sems).wait()
                else:
                    @pl.when(t > 0)
                    def _():
                        pltpu.make_async_copy(rows2.at[1],
                                              acc.at[eb.at[slot, 0, 1]],
                                              sems).wait()
                if b < bpb - 1:
                    pltpu.async_copy(source.at[eb.at[slot, b + 1, 0]],
                                     rows2.at[1 - rs], semg)

                def grp(g, _):
                    wv = wbuf[slot, b, pl.ds(g * 16, 16)]
                    for j in range(16):
                        e = g * 16 + j
                        wbc = _bcast_lane(wv, j)
                        for f in range(nf):
                            sl = pl.ds(f * 16, 16)
                            rows2[rs, e, sl] = rows2[rs, e, sl] * wbc
                    return 0
                lax.fori_loop(0, 8, grp, 0)
                pltpu.async_copy(rows2.at[rs], acc.at[eb.at[slot, b, 1]],
                                 sems, add=True)
            return 0
        lax.fori_loop(0, nbatch, batch, 0)
        pltpu.make_async_copy(rows2.at[1], acc.at[eb.at[1, 0, 1]], sems).wait()
        plsc.subcore_barrier()

        base = pl.multiple_of(s * 624, 8)
        chunks = [(0, 128), (128, 128), (256, 128), (384, 128), (512, 112)]
        for off, ln in chunks:
            pltpu.sync_copy(acc.at[pl.ds(base + off, ln)],
                            out.at[c, pl.ds(base + off, ln)])

        @pl.when(s == NTILES - 1)
        def _():
            pltpu.sync_copy(acc.at[pl.ds(9984, 16)],
                            out.at[c, pl.ds(9984, 16)])

        if scaled_out:
            def scale_chunk(off, ln):
                pltpu.sync_copy(acc.at[pl.ds(base + off, ln)],
                                rows2.at[0, pl.ds(0, ln)])
                pltpu.sync_copy(d2fs.at[pl.ds(base + off, ln)],
                                rows2.at[1, pl.ds(0, ln)])

                def rw(r, _):
                    for f in range(nf):
                        sl = pl.ds(f * 16, 16)
                        rows2[0, r, sl] = rows2[0, r, sl] * rows2[1, r, sl]
                    return 0
                lax.fori_loop(0, ln, rw, 0, unroll=8)
                pltpu.sync_copy(rows2.at[0, pl.ds(0, ln)],
                                qout.at[c, pl.ds(base + off, ln)])
            for off, ln in chunks:
                scale_chunk(off, ln)

            @pl.when(s == NTILES - 1)
            def _():
                pltpu.sync_copy(acc.at[pl.ds(9984, 16)],
                                rows2.at[0, pl.ds(0, 16)])
                pltpu.sync_copy(d2fs.at[pl.ds(9984, 16)],
                                rows2.at[1, pl.ds(0, 16)])

                def rwt(r, _):
                    for f in range(nf):
                        sl = pl.ds(f * 16, 16)
                        rows2[0, r, sl] = rows2[0, r, sl] * rows2[1, r, sl]
                    return 0
                lax.fori_loop(0, 16, rwt, 0, unroll=8)
                pltpu.sync_copy(rows2.at[0, pl.ds(0, 16)],
                                qout.at[c, pl.ds(9984, 16)])

    if scaled_out:
        fn = body
    else:
        def fn(source, e3, wmr, d2fs, out, eb, wbuf, rows2, acc,
               semb, semg, sems):
            body(source, e3, wmr, d2fs, out, None, eb, wbuf, rows2, acc,
                 semb, semg, sems)

    return functools.partial(
        pl.kernel, mesh=mesh,
        out_type=tuple(outs) if scaled_out else outs[0],
        scratch_types=scratch,
    )(fn)


_deg = _make_deg()
_hop1 = _make_hop(128, True)
_hop2 = _make_hop(128, False)
_hop3 = _make_hop(64, False)
_hop4 = _make_hop(64, False)


_BLK = 1000


def _tc_prep_body(x, h, dg, uv_o, dinvb_o, d2w_o):
    deg = dg[0] + dg[1]
    pos = deg > 0.0
    safe = jnp.where(pos, deg, 1.0)
    dinv = jnp.where(pos, lax.rsqrt(safe), 0.0)
    dinv2 = jnp.where(pos, 1.0 / safe, 0.0)
    dinvb = jnp.broadcast_to(dinv, (_BLK, D))
    uv_o[0] = dinvb * x[...]
    uv_o[1] = dinvb * h[...]
    dinvb_o[...] = dinvb
    d2w_o[...] = jnp.broadcast_to(dinv2, (_BLK, D))


def _tc_prep(x, h, degp):
    row = pl.BlockSpec((_BLK, D), lambda i: (i, 0))
    two = pl.BlockSpec((2, _BLK, D), lambda i: (0, i, 0))
    return pl.pallas_call(
        _tc_prep_body,
        grid=(N // _BLK,),
        in_specs=[row, row, pl.BlockSpec((2, _BLK, 1), lambda i: (0, i, 0))],
        out_specs=[two, row, row],
        out_shape=[jax.ShapeDtypeStruct((2, N, D), jnp.float32),
                   jax.ShapeDtypeStruct((N, D), jnp.float32),
                   jax.ShapeDtypeStruct((N, D), jnp.float32)],
    )(x, h, degp)


def _tc_gates_body(x, h, t1, t2, dinvb_r, g, b3, z_o, hr_o, uhr_o, xh_o):
    dinvb = dinvb_r[...]
    ins = (x[...], dinvb * t1[0], dinvb * t2[0],
           h[...], dinvb * t1[1], dinvb * t2[1])
    acc = jnp.zeros((_BLK, 3 * D), jnp.float32)
    for k in range(6):
        acc = acc + lax.dot_general(
            ins[k], g[k], (((1,), (0,)), ((), ())),
            preferred_element_type=jnp.float32)
    acc = acc + b3[...]
    z = jax.nn.sigmoid(acc[:, 0:D])
    r = jax.nn.sigmoid(acc[:, D:2 * D])
    hr = h[...] * r
    z_o[...] = z
    hr_o[...] = hr
    uhr_o[...] = dinvb * hr
    xh_o[...] = acc[:, 2 * D:3 * D]


def _tc_gates(x, h, t1, t2, dinvb, g, b3):
    row = pl.BlockSpec((_BLK, D), lambda i: (i, 0))
    two = pl.BlockSpec((2, _BLK, D), lambda i: (0, i, 0))
    return pl.pallas_call(
        _tc_gates_body,
        grid=(N // _BLK,),
        in_specs=[row, row, two, two, row,
                  pl.BlockSpec((6, D, 3 * D), lambda i: (0, 0, 0)),
                  pl.BlockSpec((1, 3 * D), lambda i: (0, 0))],
        out_specs=[row, row, row, row],
        out_shape=[jax.ShapeDtypeStruct((N, D), jnp.float32)] * 4,
    )(x, h, t1, t2, dinvb, g, b3)


def _tc_mid2_body(d2w, t1r, t1s_o, qr_o):
    t1s = t1r[0] + t1r[1]
    t1s_o[...] = t1s
    qr_o[...] = d2w[...] * t1s


def _tc_mid2(d2w, t1r):
    row = pl.BlockSpec((_BLK, D), lambda i: (i, 0))
    two = pl.BlockSpec((2, _BLK, D), lambda i: (0, i, 0))
    return pl.pallas_call(
        _tc_mid2_body,
        grid=(N // _BLK,),
        in_specs=[row, two],
        out_specs=[row, row],
        out_shape=[jax.ShapeDtypeStruct((N, D), jnp.float32)] * 2,
    )(d2w, t1r)


def _tc_final_body(z, h, xh, hr, t1s, t2r, dinvb_r, gh, bh, o):
    dinvb = dinvb_r[...]
    ins = (hr[...], dinvb * t1s[...], dinvb * (t2r[0] + t2r[1]))
    acc = xh[...] + bh[...]
    for k in range(3):
        acc = acc + lax.dot_general(
            ins[k], gh[k], (((1,), (0,)), ((), ())),
            preferred_element_type=jnp.float32)
    ht = jnp.tanh(acc)
    zz = z[...]
    o[...] = zz * h[...] + (1.0 - zz) * ht


def _tc_final(z, h, xh, hr, t1s, t2r, dinvb, gh, bh):
    row = pl.BlockSpec((_BLK, D), lambda i: (i, 0))
    two = pl.BlockSpec((2, _BLK, D), lambda i: (0, i, 0))
    return pl.pallas_call(
        _tc_final_body,
        grid=(N // _BLK,),
        in_specs=[row, row, row, row, row, two, row,
                  pl.BlockSpec((3, D, D), lambda i: (0, 0, 0)),
                  pl.BlockSpec((1, D), lambda i: (0, 0))],
        out_specs=row,
        out_shape=jax.ShapeDtypeStruct((N, D), jnp.float32),
    )(z, h, xh, hr, t1s, t2r, dinvb, gh, bh)


def kernel(X, edge_index, edge_weight, H, Wxz, bxz, Whz, bhz, Wxr, bxr,
           Whr, bhr, Wxh, bxh, Whh, bhh):
    src = edge_index[0]
    dst = edge_index[1]
    pad = E_PAD - E
    spread = jnp.arange(pad, dtype=src.dtype) % N
    srcr = jnp.concatenate([src, spread]).reshape(EROWS, 128)
    dstr = jnp.concatenate([dst, spread]).reshape(EROWS, 128)
    wm = jnp.where(src != dst, edge_weight, 0.0)
    wmr = jnp.concatenate(
        [wm, jnp.zeros((pad,), edge_weight.dtype)]).reshape(EROWS, 128)
    e3 = jnp.concatenate([
        jnp.stack([srcr, dstr], axis=1),
        jnp.stack([srcr + N, dstr], axis=1),
    ])

    def fold(w):
        return w[0] - w[2], -w[1], 2.0 * w[2]

    axz, bxz_, cxz = fold(Wxz)
    ahz, bhz_, chz = fold(Whz)
    axr, bxr_, cxr = fold(Wxr)
    ahr, bhr_, chr_ = fold(Whr)
    axh, bxh_, cxh = fold(Wxh)
    ahh, bhh_, chh = fold(Whh)
    g = jnp.stack([
        jnp.concatenate([axz, axr, axh], axis=1),
        jnp.concatenate([bxz_, bxr_, bxh_], axis=1),
        jnp.concatenate([cxz, cxr, cxh], axis=1),
        jnp.concatenate([ahz, ahr, jnp.zeros((D, D), jnp.float32)], axis=1),
        jnp.concatenate([bhz_, bhr_, jnp.zeros((D, D), jnp.float32)], axis=1),
        jnp.concatenate([chz, chr_, jnp.zeros((D, D), jnp.float32)], axis=1),
    ])
    b3 = jnp.concatenate([bxz + bhz, bxr + bhr, bxh]).reshape(1, 3 * D)
    gh = jnp.stack([ahh, bhh_, chh])
    bhv = bhh.reshape(1, D)

    degp = _deg(srcr, wmr)
    uv, dinvb, d2w = _tc_prep(X, H, degp.reshape(2, N, 1))
    t1, q = _hop1(uv.reshape(2 * N, D), e3, wmr, d2w)
    t2 = _hop2(q.reshape(2 * N, D), e3, wmr, d2w)
    z, hr, uhr, xh_part = _tc_gates(X, H, t1, t2, dinvb, g, b3)
    t1r_parts = _hop3(uhr, e3, wmr, d2w)
    t1s, qr = _tc_mid2(d2w, t1r_parts)
    t2r_parts = _hop4(qr, e3, wmr, d2w)
    return _tc_final(z, H, xh_part, hr, t1s, t2r_parts, dinvb, gh, bhv)


# speedup vs baseline: 13.3227x; 1.0084x over previous
import functools

import jax
import jax.numpy as jnp
from jax import lax
from jax.experimental import pallas as pl
from jax.experimental.pallas import tpu as pltpu
from jax.experimental.pallas import tpu_sc as plsc

N = 10000
E = 320000
D = 128
E_PAD = 327680
EROWS = E_PAD // 128
NTILES = 16


def _zeros16f():
    return jnp.zeros((16,), jnp.float32)


def _bcast_lane(v, j):
    return jnp.take(v, jnp.full((16,), j, jnp.int32))


def _zero_rows(rows_t, nrows, ngrp):
    def body(r, _):
        for g in range(ngrp):
            rows_t[r, pl.ds(g * 16, 16)] = _zeros16f()
        return 0
    lax.fori_loop(0, nrows, body, 0, unroll=8)


def _zero_acc_slice(acc, rows2, s):
    base = pl.multiple_of(s * 624, 8)
    for off in (0, 128, 256, 384):
        pltpu.sync_copy(rows2.at[0], acc.at[pl.ds(base + off, 128)])
    pltpu.sync_copy(rows2.at[0, pl.ds(0, 112)], acc.at[pl.ds(base + 512, 112)])

    @pl.when(s == NTILES - 1)
    def _():
        pltpu.sync_copy(rows2.at[0, pl.ds(0, 16)], acc.at[pl.ds(9984, 16)])


def _writeout(acc, out_ref, c, s):
    base = pl.multiple_of(s * 624, 8)
    pltpu.sync_copy(acc.at[pl.ds(base, 624)], out_ref.at[c, pl.ds(base, 624)])

    @pl.when(s == NTILES - 1)
    def _():
        pltpu.sync_copy(acc.at[pl.ds(9984, 16)], out_ref.at[c, pl.ds(9984, 16)])


def _make_deg():
    mesh = plsc.VectorSubcoreMesh(core_axis_name="c", subcore_axis_name="s")

    @functools.partial(
        pl.kernel, mesh=mesh,
        out_type=jax.ShapeDtypeStruct((2, N), jnp.float32),
        scratch_types=dict(
            src2=pltpu.VMEM((80, 128), jnp.int32),
            w2=pltpu.VMEM((80, 128), jnp.float32),
            z=pltpu.VMEM((640,), jnp.float32),
            sdeg=pltpu.VMEM_SHARED((N,), jnp.float32),
        ),
    )
    def deg_kernel(srcr, wmr, deg_out, src2, w2, z, sdeg):
        c = lax.axis_index("c")
        s = lax.axis_index("s")
        rb = (c * NTILES + s) * 80
        pltpu.sync_copy(srcr.at[pl.ds(rb, 80)], src2)
        pltpu.sync_copy(wmr.at[pl.ds(rb, 80)], w2)

        def zb(r, _):
            z[pl.ds(r * 16, 16)] = _zeros16f()
            return 0
        lax.fori_loop(0, 40, zb, 0)
        base = pl.multiple_of(s * 624, 8)
        pltpu.sync_copy(z.at[pl.ds(0, 624)], sdeg.at[pl.ds(base, 624)])

        @pl.when(s == NTILES - 1)
        def _():
            pltpu.sync_copy(z.at[pl.ds(0, 16)], sdeg.at[pl.ds(9984, 16)])
        plsc.subcore_barrier()

        def blk(b, _):
            pltpu.sync_copy(w2.at[b], sdeg.at[src2.at[b]], add=True)
            return 0
        lax.fori_loop(0, 80, blk, 0)
        plsc.subcore_barrier()

        @pl.when(s == 0)
        def _():
            pltpu.sync_copy(sdeg, deg_out.at[c])

    return deg_kernel


def _make_hop(width, scaled_out):
    mesh = plsc.VectorSubcoreMesh(core_axis_name="c", subcore_axis_name="s")
    full_e = width == 128
    nbatch = 10 if full_e else 5
    bpb = 16
    w = 128
    nf = w // 16
    outs = [jax.ShapeDtypeStruct((2, N, w), jnp.float32)]
    if scaled_out:
        outs.append(jax.ShapeDtypeStruct((2, N, w), jnp.float32))

    scratch = dict(
        eb=pltpu.VMEM((2, bpb, 2, 128), jnp.int32),
        wbuf=pltpu.VMEM((2, bpb, 128), jnp.float32),
        rows2=pltpu.VMEM((2, 128, w), jnp.float32),
        acc=pltpu.VMEM_SHARED((N, w), jnp.float32),
        semb=pltpu.SemaphoreType.DMA,
        semg=pltpu.SemaphoreType.DMA,
        sems=pltpu.SemaphoreType.DMA,
    )

    def body(source, e3, wmr, d2fs, out, qout, eb, wbuf, rows2, acc,
             semb, semg, sems):
        c = lax.axis_index("c")
        s = lax.axis_index("s")
        if full_e:
            rb = c * EROWS + s * 160
            wb0 = s * 160
        else:
            rb = (c * NTILES + s) * 80
            wb0 = rb

        pltpu.async_copy(e3.at[pl.ds(rb, bpb)], eb.at[0], semb)
        pltpu.async_copy(wmr.at[pl.ds(wb0, bpb)], wbuf.at[0], semb)

        def zr(r, _):
            for g in range(nf):
                rows2[0, r, pl.ds(g * 16, 16)] = _zeros16f()
            return 0
        lax.fori_loop(0, 128, zr, 0, unroll=8)
        base = pl.multiple_of(s * 624, 8)
        zchunks = [(0, 128), (128, 128), (256, 128), (384, 128), (512, 112)]
        for off, ln in zchunks:
            pltpu.async_copy(rows2.at[0, pl.ds(0, ln)],
                             acc.at[pl.ds(base + off, ln)], sems)

        @pl.when(s == NTILES - 1)
        def _():
            pltpu.async_copy(rows2.at[0, pl.ds(0, 16)],
                             acc.at[pl.ds(9984, 16)], sems)
        for off, ln in zchunks:
            pltpu.make_async_copy(rows2.at[0, pl.ds(0, ln)],
                                  acc.at[pl.ds(base + off, ln)], sems).wait()

        @pl.when(s == NTILES - 1)
        def _():
            pltpu.make_async_copy(rows2.at[0, pl.ds(0, 16)],
                                  acc.at[pl.ds(9984, 16)], sems).wait()
        plsc.subcore_barrier()

        def batch(t, _):
            slot = lax.rem(t, 2)
            pltpu.make_async_copy(e3.at[pl.ds(rb, bpb)], eb.at[slot],
                                  semb).wait()
            pltpu.make_async_copy(wmr.at[pl.ds(wb0, bpb)], wbuf.at[slot],
                                  semb).wait()

            @pl.when(t < nbatch - 1)
            def _():
                pltpu.async_copy(e3.at[pl.ds(rb + (t + 1) * bpb, bpb)],
                                 eb.at[1 - slot], semb)
                pltpu.async_copy(wmr.at[pl.ds(wb0 + (t + 1) * bpb, bpb)],
                                 wbuf.at[1 - slot], semb)

            pltpu.async_copy(source.at[eb.at[slot, 0, 0]], rows2.at[0], semg)

            for b in range(bpb):
                rs = b & 1
                pltpu.make_async_copy(source.at[eb.at[slot, b, 0]],
                                      rows2.at[rs], semg).wait()
                if b > 0:
                    pltpu.make_async_copy(rows2.at[1 - rs],
                                          acc.at[eb.at[slot, b - 1, 1]],
                                          sems).wait()
                else:
                    @pl.when(t > 0)
                    def _():
                        pltpu.make_async_copy(rows2.at[1],
                                              acc.at[eb.at[slot, 0, 1]],
                                              sems).wait()
                if b < bpb - 1:
                    pltpu.async_copy(source.at[eb.at[slot, b + 1, 0]],
                                     rows2.at[1 - rs], semg)

                def grp(g, _):
                    wv = wbuf[slot, b, pl.ds(g * 16, 16)]
                    for j in range(16):
                        e = g * 16 + j
                        wbc = _bcast_lane(wv, j)
                        for f in range(nf):
                            sl = pl.ds(f * 16, 16)
                            rows2[rs, e, sl] = rows2[rs, e, sl] * wbc
                    return 0
                lax.fori_loop(0, 8, grp, 0)
                pltpu.async_copy(rows2.at[rs], acc.at[eb.at[slot, b, 1]],
                                 sems, add=True)
            return 0
        lax.fori_loop(0, nbatch, batch, 0)
        pltpu.make_async_copy(rows2.at[1], acc.at[eb.at[1, 0, 1]], sems).wait()
        plsc.subcore_barrier()

        chunks = [(0, 128), (128, 128), (256, 128), (384, 128), (512, 112)]
        for off, ln in chunks:
            pltpu.async_copy(acc.at[pl.ds(base + off, ln)],
                             out.at[c, pl.ds(base + off, ln)], semg)

        @pl.when(s == NTILES - 1)
        def _():
            pltpu.async_copy(acc.at[pl.ds(9984, 16)],
                             out.at[c, pl.ds(9984, 16)], semg)

        if scaled_out:
            def scale_chunk(off, ln):
                pltpu.sync_copy(acc.at[pl.ds(base + off, ln)],
                                rows2.at[0, pl.ds(0, ln)])
                pltpu.sync_copy(d2fs.at[pl.ds(base + off, ln)],
                                rows2.at[1, pl.ds(0, ln)])

                def rw(r, _):
                    for f in range(nf):
                        sl = pl.ds(f * 16, 16)
                        rows2[0, r, sl] = rows2[0, r, sl] * rows2[1, r, sl]
                    return 0
                lax.fori_loop(0, ln, rw, 0, unroll=8)
                pltpu.sync_copy(rows2.at[0, pl.ds(0, ln)],
                                qout.at[c, pl.ds(base + off, ln)])
            for off, ln in chunks:
                scale_chunk(off, ln)

            @pl.when(s == NTILES - 1)
            def _():
                pltpu.sync_copy(acc.at[pl.ds(9984, 16)],
                                rows2.at[0, pl.ds(0, 16)])
                pltpu.sync_copy(d2fs.at[pl.ds(9984, 16)],
                                rows2.at[1, pl.ds(0, 16)])

                def rwt(r, _):
                    for f in range(nf):
                        sl = pl.ds(f * 16, 16)
                        rows2[0, r, sl] = rows2[0, r, sl] * rows2[1, r, sl]
                    return 0
                lax.fori_loop(0, 16, rwt, 0, unroll=8)
                pltpu.sync_copy(rows2.at[0, pl.ds(0, 16)],
                                qout.at[c, pl.ds(9984, 16)])

        for off, ln in chunks:
            pltpu.make_async_copy(acc.at[pl.ds(base + off, ln)],
                                  out.at[c, pl.ds(base + off, ln)],
                                  semg).wait()

        @pl.when(s == NTILES - 1)
        def _():
            pltpu.make_async_copy(acc.at[pl.ds(9984, 16)],
                                  out.at[c, pl.ds(9984, 16)], semg).wait()

    if scaled_out:
        fn = body
    else:
        def fn(source, e3, wmr, d2fs, out, eb, wbuf, rows2, acc,
               semb, semg, sems):
            body(source, e3, wmr, d2fs, out, None, eb, wbuf, rows2, acc,
                 semb, semg, sems)

    return functools.partial(
        pl.kernel, mesh=mesh,
        out_type=tuple(outs) if scaled_out else outs[0],
        scratch_types=scratch,
    )(fn)


_deg = _make_deg()
_hop1 = _make_hop(128, True)
_hop2 = _make_hop(128, False)
_hop3 = _make_hop(64, False)
_hop4 = _make_hop(64, False)


_BLK = 1000


def _tc_prep_body(x, h, dg, uv_o, dinvb_o, d2w_o):
    deg = dg[0] + dg[1]
    pos = deg > 0.0
    safe = jnp.where(pos, deg, 1.0)
    dinv = jnp.where(pos, lax.rsqrt(safe), 0.0)
    dinv2 = jnp.where(pos, 1.0 / safe, 0.0)
    dinvb = jnp.broadcast_to(dinv, (_BLK, D))
    uv_o[0] = dinvb * x[...]
    uv_o[1] = dinvb * h[...]
    dinvb_o[...] = dinvb
    d2w_o[...] = jnp.broadcast_to(dinv2, (_BLK, D))


def _tc_prep(x, h, degp):
    row = pl.BlockSpec((_BLK, D), lambda i: (i, 0))
    two = pl.BlockSpec((2, _BLK, D), lambda i: (0, i, 0))
    return pl.pallas_call(
        _tc_prep_body,
        grid=(N // _BLK,),
        in_specs=[row, row, pl.BlockSpec((2, _BLK, 1), lambda i: (0, i, 0))],
        out_specs=[two, row, row],
        out_shape=[jax.ShapeDtypeStruct((2, N, D), jnp.float32),
                   jax.ShapeDtypeStruct((N, D), jnp.float32),
                   jax.ShapeDtypeStruct((N, D), jnp.float32)],
    )(x, h, degp)


def _tc_gates_body(x, h, t1, t2, dinvb_r, g, b3, z_o, hr_o, uhr_o, xh_o):
    dinvb = dinvb_r[...]
    ins = (x[...], dinvb * t1[0], dinvb * t2[0],
           h[...], dinvb * t1[1], dinvb * t2[1])
    acc = jnp.zeros((_BLK, 3 * D), jnp.float32)
    for k in range(6):
        acc = acc + lax.dot_general(
            ins[k], g[k], (((1,), (0,)), ((), ())),
            preferred_element_type=jnp.float32)
    acc = acc + b3[...]
    z = jax.nn.sigmoid(acc[:, 0:D])
    r = jax.nn.sigmoid(acc[:, D:2 * D])
    hr = h[...] * r
    z_o[...] = z
    hr_o[...] = hr
    uhr_o[...] = dinvb * hr
    xh_o[...] = acc[:, 2 * D:3 * D]


def _tc_gates(x, h, t1, t2, dinvb, g, b3):
    row = pl.BlockSpec((_BLK, D), lambda i: (i, 0))
    two = pl.BlockSpec((2, _BLK, D), lambda i: (0, i, 0))
    return pl.pallas_call(
        _tc_gates_body,
        grid=(N // _BLK,),
        in_specs=[row, row, two, two, row,
                  pl.BlockSpec((6, D, 3 * D), lambda i: (0, 0, 0)),
                  pl.BlockSpec((1, 3 * D), lambda i: (0, 0))],
        out_specs=[row, row, row, row],
        out_shape=[jax.ShapeDtypeStruct((N, D), jnp.float32)] * 4,
    )(x, h, t1, t2, dinvb, g, b3)


def _tc_mid2_body(d2w, t1r, t1s_o, qr_o):
    t1s = t1r[0] + t1r[1]
    t1s_o[...] = t1s
    qr_o[...] = d2w[...] * t1s


def _tc_mid2(d2w, t1r):
    row = pl.BlockSpec((_BLK, D), lambda i: (i, 0))
    two = pl.BlockSpec((2, _BLK, D), lambda i: (0, i, 0))
    return pl.pallas_call(
        _tc_mid2_body,
        grid=(N // _BLK,),
        in_specs=[row, two],
        out_specs=[row, row],
        out_shape=[jax.ShapeDtypeStruct((N, D), jnp.float32)] * 2,
    )(d2w, t1r)


def _tc_final_body(z, h, xh, hr, t1s, t2r, dinvb_r, gh, bh, o):
    dinvb = dinvb_r[...]
    ins = (hr[...], dinvb * t1s[...], dinvb * (t2r[0] + t2r[1]))
    acc = xh[...] + bh[...]
    for k in range(3):
        acc = acc + lax.dot_general(
            ins[k], gh[k], (((1,), (0,)), ((), ())),
            preferred_element_type=jnp.float32)
    ht = jnp.tanh(acc)
    zz = z[...]
    o[...] = zz * h[...] + (1.0 - zz) * ht


def _tc_final(z, h, xh, hr, t1s, t2r, dinvb, gh, bh):
    row = pl.BlockSpec((_BLK, D), lambda i: (i, 0))
    two = pl.BlockSpec((2, _BLK, D), lambda i: (0, i, 0))
    return pl.pallas_call(
        _tc_final_body,
        grid=(N // _BLK,),
        in_specs=[row, row, row, row, row, two, row,
                  pl.BlockSpec((3, D, D), lambda i: (0, 0, 0)),
                  pl.BlockSpec((1, D), lambda i: (0, 0))],
        out_specs=row,
        out_shape=jax.ShapeDtypeStruct((N, D), jnp.float32),
    )(z, h, xh, hr, t1s, t2r, dinvb, gh, bh)


def kernel(X, edge_index, edge_weight, H, Wxz, bxz, Whz, bhz, Wxr, bxr,
           Whr, bhr, Wxh, bxh, Whh, bhh):
    src = edge_index[0]
    dst = edge_index[1]
    pad = E_PAD - E
    spread = jnp.arange(pad, dtype=src.dtype) % N
    srcr = jnp.concatenate([src, spread]).reshape(EROWS, 128)
    dstr = jnp.concatenate([dst, spread]).reshape(EROWS, 128)
    wm = jnp.where(src != dst, edge_weight, 0.0)
    wmr = jnp.concatenate(
        [wm, jnp.zeros((pad,), edge_weight.dtype)]).reshape(EROWS, 128)
    e3 = jnp.concatenate([
        jnp.stack([srcr, dstr], axis=1),
        jnp.stack([srcr + N, dstr], axis=1),
    ])

    def fold(w):
        return w[0] - w[2], -w[1], 2.0 * w[2]

    axz, bxz_, cxz = fold(Wxz)
    ahz, bhz_, chz = fold(Whz)
    axr, bxr_, cxr = fold(Wxr)
    ahr, bhr_, chr_ = fold(Whr)
    axh, bxh_, cxh = fold(Wxh)
    ahh, bhh_, chh = fold(Whh)
    g = jnp.stack([
        jnp.concatenate([axz, axr, axh], axis=1),
        jnp.concatenate([bxz_, bxr_, bxh_], axis=1),
        jnp.concatenate([cxz, cxr, cxh], axis=1),
        jnp.concatenate([ahz, ahr, jnp.zeros((D, D), jnp.float32)], axis=1),
        jnp.concatenate([bhz_, bhr_, jnp.zeros((D, D), jnp.float32)], axis=1),
        jnp.concatenate([chz, chr_, jnp.zeros((D, D), jnp.float32)], axis=1),
    ])
    b3 = jnp.concatenate([bxz + bhz, bxr + bhr, bxh]).reshape(1, 3 * D)
    gh = jnp.stack([ahh, bhh_, chh])
    bhv = bhh.reshape(1, D)

    degp = _deg(srcr, wmr)
    uv, dinvb, d2w = _tc_prep(X, H, degp.reshape(2, N, 1))
    t1, q = _hop1(uv.reshape(2 * N, D), e3, wmr, d2w)
    t2 = _hop2(q.reshape(2 * N, D), e3, wmr, d2w)
    z, hr, uhr, xh_part = _tc_gates(X, H, t1, t2, dinvb, g, b3)
    t1r_parts = _hop3(uhr, e3, wmr, d2w)
    t1s, qr = _tc_mid2(d2w, t1r_parts)
    t2r_parts = _hop4(qr, e3, wmr, d2w)
    return _tc_final(z, H, xh_part, hr, t1s, t2r_parts, dinvb, gh, bhv)


# speedup vs baseline: 13.4943x; 1.0129x over previous
import functools

import jax
import jax.numpy as jnp
from jax import lax
from jax.experimental import pallas as pl
from jax.experimental.pallas import tpu as pltpu
from jax.experimental.pallas import tpu_sc as plsc

N = 10000
E = 320000
D = 128
E_PAD = 327680
EROWS = E_PAD // 128
NTILES = 16


def _zeros16f():
    return jnp.zeros((16,), jnp.float32)


def _bcast_lane(v, j):
    return jnp.take(v, jnp.full((16,), j, jnp.int32))


def _make_deg():
    mesh = plsc.VectorSubcoreMesh(core_axis_name="c", subcore_axis_name="s")

    @functools.partial(
        pl.kernel, mesh=mesh,
        out_type=jax.ShapeDtypeStruct((2, N), jnp.float32),
        scratch_types=dict(
            src2=pltpu.VMEM((80, 128), jnp.int32),
            w2=pltpu.VMEM((80, 128), jnp.float32),
            z=pltpu.VMEM((640,), jnp.float32),
            sdeg=pltpu.VMEM_SHARED((N,), jnp.float32),
        ),
    )
    def deg_kernel(srcr, wmr, deg_out, src2, w2, z, sdeg):
        c = lax.axis_index("c")
        s = lax.axis_index("s")
        rb = (c * NTILES + s) * 80
        pltpu.sync_copy(srcr.at[pl.ds(rb, 80)], src2)
        pltpu.sync_copy(wmr.at[pl.ds(rb, 80)], w2)

        def zb(r, _):
            z[pl.ds(r * 16, 16)] = _zeros16f()
            return 0
        lax.fori_loop(0, 40, zb, 0)
        base = pl.multiple_of(s * 624, 8)
        pltpu.sync_copy(z.at[pl.ds(0, 624)], sdeg.at[pl.ds(base, 624)])

        @pl.when(s == NTILES - 1)
        def _():
            pltpu.sync_copy(z.at[pl.ds(0, 16)], sdeg.at[pl.ds(9984, 16)])
        plsc.subcore_barrier()

        def blk(b, _):
            pltpu.sync_copy(w2.at[b], sdeg.at[src2.at[b]], add=True)
            return 0
        lax.fori_loop(0, 80, blk, 0)
        plsc.subcore_barrier()

        @pl.when(s == 0)
        def _():
            pltpu.sync_copy(sdeg, deg_out.at[c])

    return deg_kernel


def _make_hop(full_e, scaled_out):
    mesh = plsc.VectorSubcoreMesh(core_axis_name="c", subcore_axis_name="s")
    nbatch = 10 if full_e else 5
    bpb = 16
    w = D
    nf = w // 16
    outs = [jax.ShapeDtypeStruct((2, N, w), jnp.float32)]
    if scaled_out:
        outs.append(jax.ShapeDtypeStruct((2, N, w), jnp.float32))

    scratch = dict(
        eb=pltpu.VMEM((2, bpb, 2, 128), jnp.int32),
        wbuf=pltpu.VMEM((2, bpb, 128), jnp.float32),
        rows2=pltpu.VMEM((2, 128, w), jnp.float32),
        acc=pltpu.VMEM_SHARED((N, w), jnp.float32),
        semb=pltpu.SemaphoreType.DMA,
        semg=pltpu.SemaphoreType.DMA,
        sems=pltpu.SemaphoreType.DMA,
    )

    def body(source, e3, wmr, d2fs, out, qout, eb, wbuf, rows2, acc,
             semb, semg, sems):
        c = lax.axis_index("c")
        s = lax.axis_index("s")
        if full_e:
            rb = c * EROWS + s * 160
            wb0 = s * 160
        else:
            rb = (c * NTILES + s) * 80
            wb0 = rb

        pltpu.async_copy(e3.at[pl.ds(rb, bpb)], eb.at[0], semb)
        pltpu.async_copy(wmr.at[pl.ds(wb0, bpb)], wbuf.at[0], semb)

        def zr(r, _):
            for g in range(nf):
                rows2[0, r, pl.ds(g * 16, 16)] = _zeros16f()
            return 0
        lax.fori_loop(0, 128, zr, 0, unroll=8)
        base = pl.multiple_of(s * 624, 8)
        zchunks = [(0, 128), (128, 128), (256, 128), (384, 128), (512, 112)]
        for off, ln in zchunks:
            pltpu.async_copy(rows2.at[0, pl.ds(0, ln)],
                             acc.at[pl.ds(base + off, ln)], sems)

        @pl.when(s == NTILES - 1)
        def _():
            pltpu.async_copy(rows2.at[0, pl.ds(0, 16)],
                             acc.at[pl.ds(9984, 16)], sems)
        for off, ln in zchunks:
            pltpu.make_async_copy(rows2.at[0, pl.ds(0, ln)],
                                  acc.at[pl.ds(base + off, ln)], sems).wait()

        @pl.when(s == NTILES - 1)
        def _():
            pltpu.make_async_copy(rows2.at[0, pl.ds(0, 16)],
                                  acc.at[pl.ds(9984, 16)], sems).wait()
        plsc.subcore_barrier()

        def batch(t, _):
            slot = lax.rem(t, 2)
            pltpu.make_async_copy(e3.at[pl.ds(rb, bpb)], eb.at[slot],
                                  semb).wait()
            pltpu.make_async_copy(wmr.at[pl.ds(wb0, bpb)], wbuf.at[slot],
                                  semb).wait()

            @pl.when(t < nbatch - 1)
            def _():
                pltpu.async_copy(e3.at[pl.ds(rb + (t + 1) * bpb, bpb)],
                                 eb.at[1 - slot], semb)
                pltpu.async_copy(wmr.at[pl.ds(wb0 + (t + 1) * bpb, bpb)],
                                 wbuf.at[1 - slot], semb)

            pltpu.async_copy(source.at[eb.at[slot, 0, 0]], rows2.at[0], semg)

            for b in range(bpb):
                rs = b & 1
                pltpu.make_async_copy(source.at[eb.at[slot, b, 0]],
                                      rows2.at[rs], semg).wait()
                if b > 0:
                    pltpu.make_async_copy(rows2.at[1 - rs],
                                          acc.at[eb.at[slot, b - 1, 1]],
                                          sems).wait()
                else:
                    @pl.when(t > 0)
                    def _():
                        pltpu.make_async_copy(rows2.at[1],
                                              acc.at[eb.at[slot, 0, 1]],
                                              sems).wait()
                if b < bpb - 1:
                    pltpu.async_copy(source.at[eb.at[slot, b + 1, 0]],
                                     rows2.at[1 - rs], semg)

                def grp(g, _):
                    wv = wbuf[slot, b, pl.ds(g * 16, 16)]
                    for j in range(16):
                        e = g * 16 + j
                        wbc = _bcast_lane(wv, j)
                        for f in range(nf):
                            sl = pl.ds(f * 16, 16)
                            rows2[rs, e, sl] = rows2[rs, e, sl] * wbc
                    return 0
                lax.fori_loop(0, 8, grp, 0)
                pltpu.async_copy(rows2.at[rs], acc.at[eb.at[slot, b, 1]],
                                 sems, add=True)
            return 0
        lax.fori_loop(0, nbatch, batch, 0)
        pltpu.make_async_copy(rows2.at[1], acc.at[eb.at[1, 0, 1]], sems).wait()
        plsc.subcore_barrier()

        chunks = [(0, 128), (128, 128), (256, 128), (384, 128), (512, 112)]
        for off, ln in chunks:
            pltpu.async_copy(acc.at[pl.ds(base + off, ln)],
                             out.at[c, pl.ds(base + off, ln)], semg)

        @pl.when(s == NTILES - 1)
        def _():
            pltpu.async_copy(acc.at[pl.ds(9984, 16)],
                             out.at[c, pl.ds(9984, 16)], semg)

        if scaled_out:
            def scale_chunk(off, ln):
                pltpu.sync_copy(acc.at[pl.ds(base + off, ln)],
                                rows2.at[0, pl.ds(0, ln)])
                pltpu.sync_copy(d2fs.at[pl.ds(base + off, ln)],
                                rows2.at[1, pl.ds(0, ln)])

                def rw(r, _):
                    for f in range(nf):
                        sl = pl.ds(f * 16, 16)
                        rows2[0, r, sl] = rows2[0, r, sl] * rows2[1, r, sl]
                    return 0
                lax.fori_loop(0, ln, rw, 0, unroll=8)
                pltpu.sync_copy(rows2.at[0, pl.ds(0, ln)],
                                qout.at[c, pl.ds(base + off, ln)])
            for off, ln in chunks:
                scale_chunk(off, ln)

            @pl.when(s == NTILES - 1)
            def _():
                pltpu.sync_copy(acc.at[pl.ds(9984, 16)],
                                rows2.at[0, pl.ds(0, 16)])
                pltpu.sync_copy(d2fs.at[pl.ds(9984, 16)],
                                rows2.at[1, pl.ds(0, 16)])

                def rwt(r, _):
                    for f in range(nf):
                        sl = pl.ds(f * 16, 16)
                        rows2[0, r, sl] = rows2[0, r, sl] * rows2[1, r, sl]
                    return 0
                lax.fori_loop(0, 16, rwt, 0, unroll=8)
                pltpu.sync_copy(rows2.at[0, pl.ds(0, 16)],
                                qout.at[c, pl.ds(9984, 16)])

        for off, ln in chunks:
            pltpu.make_async_copy(acc.at[pl.ds(base + off, ln)],
                                  out.at[c, pl.ds(base + off, ln)],
                                  semg).wait()

        @pl.when(s == NTILES - 1)
        def _():
            pltpu.make_async_copy(acc.at[pl.ds(9984, 16)],
                                  out.at[c, pl.ds(9984, 16)], semg).wait()

    if scaled_out:
        fn = body
    else:
        def fn(source, e3, wmr, d2fs, out, eb, wbuf, rows2, acc,
               semb, semg, sems):
            body(source, e3, wmr, d2fs, out, None, eb, wbuf, rows2, acc,
                 semb, semg, sems)

    return functools.partial(
        pl.kernel, mesh=mesh,
        out_type=tuple(outs) if scaled_out else outs[0],
        scratch_types=scratch,
    )(fn)


_deg = _make_deg()
_hop1 = _make_hop(True, True)
_hop2 = _make_hop(True, False)
_hop3 = _make_hop(False, False)
_hop4 = _hop3


_BLK = 1000


def _tc_prep_body(x, h, dg, uv_o, dinvb_o, d2w_o):
    deg = dg[0] + dg[1]
    pos = deg > 0.0
    safe = jnp.where(pos, deg, 1.0)
    dinv = jnp.where(pos, lax.rsqrt(safe), 0.0)
    dinv2 = jnp.where(pos, 1.0 / safe, 0.0)
    dinvb = jnp.broadcast_to(dinv, (_BLK, D))
    uv_o[0] = dinvb * x[...]
    uv_o[1] = dinvb * h[...]
    dinvb_o[...] = dinvb
    d2w_o[...] = jnp.broadcast_to(dinv2, (_BLK, D))


def _tc_prep(x, h, degp):
    row = pl.BlockSpec((_BLK, D), lambda i: (i, 0))
    two = pl.BlockSpec((2, _BLK, D), lambda i: (0, i, 0))
    return pl.pallas_call(
        _tc_prep_body,
        grid=(N // _BLK,),
        in_specs=[row, row, pl.BlockSpec((2, _BLK, 1), lambda i: (0, i, 0))],
        out_specs=[two, row, row],
        out_shape=[jax.ShapeDtypeStruct((2, N, D), jnp.float32),
                   jax.ShapeDtypeStruct((N, D), jnp.float32),
                   jax.ShapeDtypeStruct((N, D), jnp.float32)],
    )(x, h, degp)


def _tc_gates_body(x, h, t1, t2, dinvb_r, g, b3, z_o, hr_o, uhr_o, xh_o):
    dinvb = dinvb_r[...]
    ins = (x[...], dinvb * t1[0], dinvb * t2[0],
           h[...], dinvb * t1[1], dinvb * t2[1])
    acc = jnp.zeros((_BLK, 3 * D), jnp.float32)
    for k in range(6):
        acc = acc + lax.dot_general(
            ins[k], g[k], (((1,), (0,)), ((), ())),
            preferred_element_type=jnp.float32)
    acc = acc + b3[...]
    z = jax.nn.sigmoid(acc[:, 0:D])
    r = jax.nn.sigmoid(acc[:, D:2 * D])
    hr = h[...] * r
    z_o[...] = z
    hr_o[...] = hr
    uhr_o[...] = dinvb * hr
    xh_o[...] = acc[:, 2 * D:3 * D]


def _tc_gates(x, h, t1, t2, dinvb, g, b3):
    row = pl.BlockSpec((_BLK, D), lambda i: (i, 0))
    two = pl.BlockSpec((2, _BLK, D), lambda i: (0, i, 0))
    return pl.pallas_call(
        _tc_gates_body,
        grid=(N // _BLK,),
        in_specs=[row, row, two, two, row,
                  pl.BlockSpec((6, D, 3 * D), lambda i: (0, 0, 0)),
                  pl.BlockSpec((1, 3 * D), lambda i: (0, 0))],
        out_specs=[row, row, row, row],
        out_shape=[jax.ShapeDtypeStruct((N, D), jnp.float32)] * 4,
    )(x, h, t1, t2, dinvb, g, b3)


def _tc_mid2_body(d2w, t1r, t1s_o, qr_o):
    t1s = t1r[0] + t1r[1]
    t1s_o[...] = t1s
    qr_o[...] = d2w[...] * t1s


def _tc_mid2(d2w, t1r):
    row = pl.BlockSpec((_BLK, D), lambda i: (i, 0))
    two = pl.BlockSpec((2, _BLK, D), lambda i: (0, i, 0))
    return pl.pallas_call(
        _tc_mid2_body,
        grid=(N // _BLK,),
        in_specs=[row, two],
        out_specs=[row, row],
        out_shape=[jax.ShapeDtypeStruct((N, D), jnp.float32)] * 2,
    )(d2w, t1r)


def _tc_final_body(z, h, xh, hr, t1s, t2r, dinvb_r, gh, bh, o):
    dinvb = dinvb_r[...]
    ins = (hr[...], dinvb * t1s[...], dinvb * (t2r[0] + t2r[1]))
    acc = xh[...] + bh[...]
    for k in range(3):
        acc = acc + lax.dot_general(
            ins[k], gh[k], (((1,), (0,)), ((), ())),
            preferred_element_type=jnp.float32)
    ht = jnp.tanh(acc)
    zz = z[...]
    o[...] = zz * h[...] + (1.0 - zz) * ht


def _tc_final(z, h, xh, hr, t1s, t2r, dinvb, gh, bh):
    row = pl.BlockSpec((_BLK, D), lambda i: (i, 0))
    two = pl.BlockSpec((2, _BLK, D), lambda i: (0, i, 0))
    return pl.pallas_call(
        _tc_final_body,
        grid=(N // _BLK,),
        in_specs=[row, row, row, row, row, two, row,
                  pl.BlockSpec((3, D, D), lambda i: (0, 0, 0)),
                  pl.BlockSpec((1, D), lambda i: (0, 0))],
        out_specs=row,
        out_shape=jax.ShapeDtypeStruct((N, D), jnp.float32),
    )(z, h, xh, hr, t1s, t2r, dinvb, gh, bh)


def kernel(X, edge_index, edge_weight, H, Wxz, bxz, Whz, bhz, Wxr, bxr,
           Whr, bhr, Wxh, bxh, Whh, bhh):
    src = edge_index[0]
    dst = edge_index[1]
    pad = E_PAD - E
    spread = jnp.arange(pad, dtype=src.dtype) % N
    srcr = jnp.concatenate([src, spread]).reshape(EROWS, 128)
    dstr = jnp.concatenate([dst, spread]).reshape(EROWS, 128)
    wm = jnp.where(src != dst, edge_weight, 0.0)
    wmr = jnp.concatenate(
        [wm, jnp.zeros((pad,), edge_weight.dtype)]).reshape(EROWS, 128)
    e3 = jnp.concatenate([
        jnp.stack([srcr, dstr], axis=1),
        jnp.stack([srcr + N, dstr], axis=1),
    ])

    def fold(w):
        return w[0] - w[2], -w[1], 2.0 * w[2]

    axz, bxz_, cxz = fold(Wxz)
    ahz, bhz_, chz = fold(Whz)
    axr, bxr_, cxr = fold(Wxr)
    ahr, bhr_, chr_ = fold(Whr)
    axh, bxh_, cxh = fold(Wxh)
    ahh, bhh_, chh = fold(Whh)
    g = jnp.stack([
        jnp.concatenate([axz, axr, axh], axis=1),
        jnp.concatenate([bxz_, bxr_, bxh_], axis=1),
        jnp.concatenate([cxz, cxr, cxh], axis=1),
        jnp.concatenate([ahz, ahr, jnp.zeros((D, D), jnp.float32)], axis=1),
        jnp.concatenate([bhz_, bhr_, jnp.zeros((D, D), jnp.float32)], axis=1),
        jnp.concatenate([chz, chr_, jnp.zeros((D, D), jnp.float32)], axis=1),
    ])
    b3 = jnp.concatenate([bxz + bhz, bxr + bhr, bxh]).reshape(1, 3 * D)
    gh = jnp.stack([ahh, bhh_, chh])
    bhv = bhh.reshape(1, D)

    degp = _deg(srcr, wmr)
    uv, dinvb, d2w = _tc_prep(X, H, degp.reshape(2, N, 1))
    t1, q = _hop1(uv.reshape(2 * N, D), e3, wmr, d2w)
    t2 = _hop2(q.reshape(2 * N, D), e3, wmr, d2w)
    z, hr, uhr, xh_part = _tc_gates(X, H, t1, t2, dinvb, g, b3)
    t1r_parts = _hop3(uhr, e3, wmr, d2w)
    t1s, qr = _tc_mid2(d2w, t1r_parts)
    t2r_parts = _hop4(qr, e3, wmr, d2w)
    return _tc_final(z, H, xh_part, hr, t1s, t2r_parts, dinvb, gh, bhv)
